# baseline (speedup 1.0000x reference)
import functools

import jax
import jax.numpy as jnp
from jax import lax
from jax.experimental import pallas as pl
from jax.experimental.pallas import tpu as pltpu
from jax.experimental.pallas import tpu_sc as plsc

_F = 256
_L = 32
_NC = 2
_NS = 16
_NW = _NC * _NS
_CPW = _F // _NW


def _iota16():
    return lax.broadcasted_iota(jnp.int32, (16,), 0)


def _full16(v):
    return jnp.zeros((16,), jnp.int32) + v


def _load_col(tab, c):
    lo = plsc.load_gather(tab, [_iota16(), _full16(c)])
    hi = plsc.load_gather(tab, [_iota16() + 16, _full16(c)])
    return lo, hi


def _sc_body(w_hbm, o_hbm, tab_v, col_v, rows_v, sem0, sem1):
    wid = lax.axis_index("s") * _NC + lax.axis_index("c")
    c0 = wid * _CPW

    for s in range(3):
        pltpu.sync_copy(w_hbm.at[s, pl.ds(0, _L)], tab_v.at[s])

    def stage_col(sec, c):
        lo, hi = _load_col(tab_v.at[sec], c)
        col_v[pl.ds(0, 16)] = lo
        col_v[pl.ds(16, 16)] = hi
        return lo, hi

    def build0(b, c):
        lo, hi = _load_col(tab_v.at[0], c)

        def bo(d0, carry):
            def bi(d1, _):
                rows_v[b, d0, d1, pl.ds(0, 16)] = lo
                rows_v[b, d0, d1, pl.ds(16, 16)] = hi
                return _
            return lax.fori_loop(0, _L, bi, carry, unroll=8)

        lax.fori_loop(0, _L, bo, 0)

    def build1(b, c):
        stage_col(1, c)

        def bo(d1, carry):
            v = plsc.load_gather(col_v, [_full16(d1)])

            def bi(d0, _):
                rows_v[b, d0, d1, pl.ds(0, 16)] = v
                rows_v[b, d0, d1, pl.ds(16, 16)] = v
                return _
            return lax.fori_loop(0, _L, bi, carry, unroll=8)

        lax.fori_loop(0, _L, bo, 0)

    def build2(b, c):
        stage_col(2, c)

        def bo(d0, carry):
            v = plsc.load_gather(col_v, [_full16(d0)])

            def bi(d1, _):
                rows_v[b, d0, d1, pl.ds(0, 16)] = v
                rows_v[b, d0, d1, pl.ds(16, 16)] = v
                return _
            return lax.fori_loop(0, _L, bi, carry, unroll=8)

        lax.fori_loop(0, _L, bo, 0)

    builders = (build0, build1, build2)
    sems = (sem0, sem1)
    pending = [None, None]
    t = 0
    for s in range(3):
        for k in range(_CPW):
            b = t % 2
            if pending[b] is not None:
                pending[b].wait()
            c = c0 + k
            builders[s](b, c)
            h = pltpu.async_copy(rows_v.at[b], o_hbm.at[s * _F + c], sems[b])
            pending[b] = h
            t += 1
    for b in range(2):
        if pending[b] is not None:
            pending[b].wait()


def kernel(x, W0, W1, W2):
    del x
    w = jnp.stack([W0, W1, W2])
    mesh = plsc.VectorSubcoreMesh(core_axis_name="c", subcore_axis_name="s")
    k = functools.partial(
        pl.kernel,
        mesh=mesh,
        compiler_params=pltpu.CompilerParams(
            use_tc_tiling_on_sc=False, needs_layout_passes=False),
        out_type=jax.ShapeDtypeStruct((3 * _F, _L, _L, _L), jnp.float32),
        scratch_types=[
            pltpu.VMEM((3, _L, _F), jnp.float32),
            pltpu.VMEM((_L,), jnp.float32),
            pltpu.VMEM((2, _L, _L, _L), jnp.float32),
            pltpu.SemaphoreType.DMA,
            pltpu.SemaphoreType.DMA,
        ],
    )(_sc_body)
    return k(w)


# speedup vs baseline: 1.0035x; 1.0035x over previous
import functools

import jax
import jax.numpy as jnp
from jax import lax
from jax.experimental import pallas as pl
from jax.experimental.pallas import tpu as pltpu
from jax.experimental.pallas import tpu_sc as plsc

_F = 256
_L = 32
_ROW = _L * _L * _L
_NC = 2
_NS = 16
_NW = _NC * _NS
_CPW = _F // _NW


def _iota16():
    return lax.broadcasted_iota(jnp.int32, (16,), 0)


def _full16(v):
    return jnp.zeros((16,), jnp.int32) + v


def _load_col(tab, c):
    lo = plsc.load_gather(tab, [_iota16(), _full16(c)])
    hi = plsc.load_gather(tab, [_iota16() + 16, _full16(c)])
    return lo, hi


def _sc_body(w_hbm, o_hbm, tab_v, col_v, rows_v, sem0, sem1):
    wid = lax.axis_index("s") * _NC + lax.axis_index("c")
    c0 = wid * _CPW

    for s in range(3):
        pltpu.sync_copy(w_hbm.at[s, pl.ds(0, _L)], tab_v.at[s])

    def stage_col(sec, c):
        lo, hi = _load_col(tab_v.at[sec], c)
        col_v[pl.ds(0, 16)] = lo
        col_v[pl.ds(16, 16)] = hi

    def build0(b, c):
        lo, hi = _load_col(tab_v.at[0], c)

        def bi(p, _):
            rows_v[b, pl.ds(p * 32, 16)] = lo
            rows_v[b, pl.ds(p * 32 + 16, 16)] = hi
            return _

        lax.fori_loop(0, _L * _L, bi, 0, unroll=8)

    def _build_splat(b, sec, c, blk):
        stage_col(sec, c)

        def bo(j, carry):
            v = plsc.load_gather(col_v, [_full16(j)])

            def bi(p, _):
                rows_v[b, pl.ds(j * blk + p * 32, 16)] = v
                rows_v[b, pl.ds(j * blk + p * 32 + 16, 16)] = v
                return _
            return lax.fori_loop(0, blk // 32, bi, carry, unroll=8)

        lax.fori_loop(0, _L, bo, 0)

    def build1(b, c):
        stage_col(1, c)

        def bo(j, carry):
            v = plsc.load_gather(col_v, [_full16(j)])

            def bi(d0, _):
                rows_v[b, pl.ds(d0 * 1024 + j * 32, 16)] = v
                rows_v[b, pl.ds(d0 * 1024 + j * 32 + 16, 16)] = v
                return _
            return lax.fori_loop(0, _L, bi, carry, unroll=8)

        lax.fori_loop(0, _L, bo, 0)

    def build2(b, c):
        _build_splat(b, 2, c, 1024)

    builders = (build0, build1, build2)
    sems = (sem0, sem1)
    pending = [None, None]
    t = 0
    for s in range(3):
        for k in range(_CPW):
            b = t % 2
            if pending[b] is not None:
                pending[b].wait()
            c = c0 + k
            builders[s](b, c)
            r = s * _F + c
            h = pltpu.async_copy(rows_v.at[b], o_hbm.at[pl.ds(r * _ROW, _ROW)],
                                 sems[b])
            pending[b] = h
            t += 1
    for b in range(2):
        if pending[b] is not None:
            pending[b].wait()


def kernel(x, W0, W1, W2):
    del x
    w = jnp.stack([W0, W1, W2])
    mesh = plsc.VectorSubcoreMesh(core_axis_name="c", subcore_axis_name="s")
    k = functools.partial(
        pl.kernel,
        mesh=mesh,
        compiler_params=pltpu.CompilerParams(
            use_tc_tiling_on_sc=False, needs_layout_passes=False),
        out_type=jax.ShapeDtypeStruct((3 * _F * _ROW,), jnp.float32),
        scratch_types=[
            pltpu.VMEM((3, _L, _F), jnp.float32),
            pltpu.VMEM((_L,), jnp.float32),
            pltpu.VMEM((2, _ROW), jnp.float32),
            pltpu.SemaphoreType.DMA,
            pltpu.SemaphoreType.DMA,
        ],
    )(_sc_body)
    return k(w).reshape(3 * _F, _L, _L, _L)


# speedup vs baseline: 1.0050x; 1.0015x over previous
import functools

import jax
import jax.numpy as jnp
from jax import lax
from jax.experimental import pallas as pl
from jax.experimental.pallas import tpu as pltpu
from jax.experimental.pallas import tpu_sc as plsc

_F = 256
_L = 32
_ROW = _L * _L * _L
_NC = 2
_NS = 16
_NW = _NC * _NS
_CPW = _F // _NW


def _iota16():
    return lax.broadcasted_iota(jnp.int32, (16,), 0)


def _full16(v):
    return jnp.zeros((16,), jnp.int32) + v


def _sc_body(w_hbm, o_hbm, tab_v, col_v, rows_v, sem0, sem1, semt):
    wid = lax.axis_index("s") * _NC + lax.axis_index("c")
    c0 = wid * _CPW

    handles = []
    for s in range(3):
        for r in range(_L):
            handles.append(pltpu.async_copy(
                w_hbm.at[pl.ds((s * 50 + r) * _F, _F)],
                tab_v.at[pl.ds((s * _L + r) * _F, _F)], semt))
    for h in handles:
        h.wait()

    def load_col(sec, c):
        base = sec * _L * _F + c
        lo = plsc.load_gather(tab_v, [_iota16() * _F + base])
        hi = plsc.load_gather(tab_v, [(_iota16() + 16) * _F + base])
        return lo, hi

    def stage_col(sec, c):
        lo, hi = load_col(sec, c)
        col_v[pl.ds(0, 16)] = lo
        col_v[pl.ds(16, 16)] = hi

    def build0(b, c):
        lo, hi = load_col(0, c)

        def bi(p, _):
            rows_v[pl.ds(b * _ROW + p * 32, 16)] = lo
            rows_v[pl.ds(b * _ROW + p * 32 + 16, 16)] = hi
            return _

        lax.fori_loop(0, _L * _L, bi, 0, unroll=8)

    def build1(b, c):
        stage_col(1, c)

        def bo(j, carry):
            v = plsc.load_gather(col_v, [_full16(j)])

            def bi(d0, _):
                rows_v[pl.ds(b * _ROW + d0 * 1024 + j * 32, 16)] = v
                rows_v[pl.ds(b * _ROW + d0 * 1024 + j * 32 + 16, 16)] = v
                return _
            return lax.fori_loop(0, _L, bi, carry, unroll=8)

        lax.fori_loop(0, _L, bo, 0)

    def build2(b, c):
        stage_col(2, c)

        def bo(j, carry):
            v = plsc.load_gather(col_v, [_full16(j)])

            def bi(p, _):
                rows_v[pl.ds(b * _ROW + j * 1024 + p * 32, 16)] = v
                rows_v[pl.ds(b * _ROW + j * 1024 + p * 32 + 16, 16)] = v
                return _
            return lax.fori_loop(0, _L, bi, carry, unroll=8)

        lax.fori_loop(0, _L, bo, 0)

    builders = (build0, build1, build2)
    sems = (sem0, sem1)
    pending = [None, None]
    t = 0
    for s in range(3):
        for k in range(_CPW):
            b = t % 2
            if pending[b] is not None:
                pending[b].wait()
            c = c0 + k
            builders[s](b, c)
            r = s * _F + c
            h = pltpu.async_copy(rows_v.at[pl.ds(b * _ROW, _ROW)],
                                 o_hbm.at[pl.ds(r * _ROW, _ROW)], sems[b])
            pending[b] = h
            t += 1
    for b in range(2):
        if pending[b] is not None:
            pending[b].wait()


def kernel(x, W0, W1, W2):
    del x
    w = jnp.stack([W0, W1, W2]).reshape(-1)
    mesh = plsc.VectorSubcoreMesh(core_axis_name="c", subcore_axis_name="s")
    k = functools.partial(
        pl.kernel,
        mesh=mesh,
        compiler_params=pltpu.CompilerParams(needs_layout_passes=False),
        out_type=jax.ShapeDtypeStruct((3 * _F * _ROW,), jnp.float32),
        scratch_types=[
            pltpu.VMEM((3 * _L * _F,), jnp.float32),
            pltpu.VMEM((_L,), jnp.float32),
            pltpu.VMEM((2 * _ROW,), jnp.float32),
            pltpu.SemaphoreType.DMA,
            pltpu.SemaphoreType.DMA,
            pltpu.SemaphoreType.DMA,
        ],
    )(_sc_body)
    return k(w).reshape(3 * _F, _L, _L, _L)


# speedup vs baseline: 12.2736x; 12.2122x over previous
import jax
import jax.numpy as jnp
from jax.experimental import pallas as pl

_F = 256
_L = 32
_B0 = 8


def _body(w_ref, w2_ref, o_ref):
    blk = (_B0, _L, _L, _F)
    w0 = w_ref[0, :_L, :]
    w1 = w_ref[1, :_L, :]
    w2 = w2_ref[0]
    o_ref[:, :, :, 0:_F] = jnp.broadcast_to(w0[None, None, :, :], blk)
    o_ref[:, :, :, _F:2 * _F] = jnp.broadcast_to(w1[None, :, None, :], blk)
    o_ref[:, :, :, 2 * _F:3 * _F] = jnp.broadcast_to(w2[:, None, None, :], blk)


def kernel(x, W0, W1, W2):
    del x
    w = jnp.stack([W0, W1, W2])
    out = pl.pallas_call(
        _body,
        grid=(_L // _B0,),
        in_specs=[pl.BlockSpec((3, 50, _F), lambda j: (0, 0, 0)),
                  pl.BlockSpec((1, _B0, _F), lambda j: (2, j, 0))],
        out_specs=pl.BlockSpec((_B0, _L, _L, 3 * _F), lambda j: (j, 0, 0, 0)),
        out_shape=jax.ShapeDtypeStruct((_L, _L, _L, 3 * _F), jnp.float32),
    )(w, w)
    return jnp.moveaxis(out, -1, 0)
